# baseline (speedup 1.0000x reference)
import functools

import jax
import jax.numpy as jnp
from jax import lax
from jax.experimental import pallas as pl
from jax.experimental.pallas import tpu as pltpu
from jax.experimental.pallas import tpu_sc as plsc

VOCAB = 1000000
EMBED = 64
BATCH = 16384

NBLK = (VOCAB + 127) // 128
_info = plsc.get_sparse_core_info()
_NC, _NS = _info.num_cores, _info.num_subcores
NW = _NC * _NS
BLK_PER_W = NBLK // NW
EXTRA = NBLK % NW

_mesh = plsc.VectorSubcoreMesh(core_axis_name="c", subcore_axis_name="s")


@functools.partial(
    pl.kernel,
    mesh=_mesh,
    out_type=jax.ShapeDtypeStruct((BATCH * EMBED,), jnp.float32),
    scratch_types=[
        pltpu.VMEM((BATCH + 16,), jnp.int32),
        pltpu.VMEM((BATCH + 16,), jnp.int32),
        pltpu.VMEM((BATCH + 16,), jnp.int32),
        pltpu.VMEM((BATCH + 16,), jnp.int32),
        [pltpu.VMEM((EMBED, 256), jnp.float32) for _ in range(3)],
        pltpu.VMEM((32 * EMBED,), jnp.float32),
        [pltpu.SemaphoreType.DMA for _ in range(3)],
        pltpu.SemaphoreType.DMA,
    ],
    compiler_params=pltpu.CompilerParams(needs_layout_passes=False),
)
def _gather_kernel(idx_hbm, table_hbm, tail_hbm, out_hbm, idx_v, wl_v, wl_b,
                   w2_b, slabs, ring_v, sems, sem_o):
    w2_v = idx_v
    wid = lax.axis_index("s") * _NC + lax.axis_index("c")
    jstart = BLK_PER_W * wid + jnp.minimum(wid, EXTRA)
    jcount = jnp.where(wid < EXTRA, BLK_PER_W + 1, BLK_PER_W)
    jend = jstart + jcount
    iota = lax.iota(jnp.int32, 16)

    pltpu.sync_copy(idx_hbm, idx_v.at[pl.ds(0, BATCH)])

    def sel_body(g, num):
        vv = idx_v[pl.ds(g * 16, 16)]
        blk = lax.shift_right_logical(vv, 7)
        m = (blk >= jstart) & (blk < jend)
        mi = m.astype(jnp.int32)
        incl = plsc.cumsum(mi)
        pos = jnp.where(m, num + incl - mi, BATCH + 8)
        plsc.store_scatter(wl_v, [pos], vv)
        plsc.store_scatter(wl_b, [pos], g * 16 + iota)
        return num + incl[15]

    n = lax.fori_loop(0, BATCH // 16, sel_body, jnp.int32(0))
    ngrp = lax.shift_right_logical(n + 15, 4)

    bufs = [(wl_v, wl_b, w2_v, w2_b), (w2_v, w2_b, wl_v, wl_b)]
    for bit in range(8):
        sv, sb, dv, db = bufs[bit % 2]

        def mk_scan(want_one):
            def scan(g, pos):
                vv = sv[pl.ds(g * 16, 16)]
                bb = sb[pl.ds(g * 16, 16)]
                key = lax.shift_right_logical(
                    lax.shift_right_logical(vv, 7) - jstart, bit) & 1
                m = (key == want_one) & ((g * 16 + iota) < n)
                mi = m.astype(jnp.int32)
                incl = plsc.cumsum(mi)
                dst = jnp.where(m, pos + incl - mi, BATCH + 8)
                plsc.store_scatter(dv, [dst], vv)
                plsc.store_scatter(db, [dst], bb)
                return pos + incl[15]
            return scan

        z = lax.fori_loop(0, ngrp, mk_scan(0), jnp.int32(0))
        lax.fori_loop(0, ngrp, mk_scan(1), z)

    def start_fetch(j, slab, sem):
        def full(_):
            base = jnp.minimum(j, NBLK - 3) * 128
            pltpu.async_copy(table_hbm.at[:, pl.ds(base, 256)], slab, sem)
            return 0

        def tail(_):
            pltpu.async_copy(tail_hbm, slab, sem)
            return 0

        lax.cond(j >= NBLK - 2, tail, full, 0)

    def wait_fetch(j, slab, sem):
        base = jnp.minimum(j, NBLK - 3) * 128
        pltpu.make_async_copy(
            table_hbm.at[:, pl.ds(base, 256)], slab, sem).wait()

    def process_block(j, p0, slab):
        base = j * 128

        def wcond(p):
            vv = wl_v[pl.ds(p, 16)]
            blk = lax.shift_right_logical(vv[0], 7)
            return (p < n) & (blk >= j) & (blk < j + 2)

        def wbody(p):
            vv = wl_v[pl.ds(p, 16)]
            bb = wl_b[pl.ds(p, 16)]
            c = vv[0] - base
            b0 = bb[0]
            cvec = jnp.full((16,), 0, jnp.int32) + c
            slot = p & 31

            @pl.when(p >= 32)
            def _():
                pltpu.make_async_copy(
                    ring_v.at[pl.ds(slot * EMBED, EMBED)],
                    out_hbm.at[pl.ds(b0 * EMBED, EMBED)], sem_o).wait()

            for g2 in range(EMBED // 16):
                vals = plsc.load_gather(slab, [iota + g2 * 16, cvec])
                ring_v[pl.ds(slot * EMBED + g2 * 16, 16)] = vals
            pltpu.async_copy(
                ring_v.at[pl.ds(slot * EMBED, EMBED)],
                out_hbm.at[pl.ds(b0 * EMBED, EMBED)], sem_o)
            return p + 1

        return lax.while_loop(wcond, wbody, p0)

    NSLAB = len(slabs)
    for s in range(NSLAB):
        start_fetch(jstart + 2 * s, slabs[s], sems[s])

    def round_body(r, p):
        for s in range(NSLAB):
            j = jstart + 2 * (r * NSLAB + s)
            wait_fetch(j, slabs[s], sems[s])
            p = process_block(j, p, slabs[s])
            start_fetch(j + 2 * NSLAB, slabs[s], sems[s])
        return p

    npairs = lax.shift_right_logical(jcount + 1, 1)
    nrounds = (npairs + NSLAB - 1) // NSLAB
    p_end = lax.fori_loop(0, nrounds, round_body, jnp.int32(0))
    for s in range(NSLAB):
        wait_fetch(jstart + 2 * (nrounds * NSLAB + s), slabs[s], sems[s])

    def drain(r):
        pltpu.make_async_copy(
            ring_v.at[pl.ds(0, EMBED)],
            out_hbm.at[pl.ds(0, EMBED)], sem_o).wait()
        return r + 1

    lax.while_loop(lambda r: r < jnp.minimum(p_end, 32), drain, jnp.int32(0))


def kernel(centre_words, input_table):
    idx = centre_words.astype(jnp.int32)
    tail = jnp.pad(input_table[(NBLK - 2) * 128:].T, ((0, 0), (0, 64)))
    flat = _gather_kernel(idx, input_table.T, tail)
    return flat.reshape(BATCH, EMBED)


# speedup vs baseline: 1.0995x; 1.0995x over previous
import functools

import jax
import jax.numpy as jnp
from jax import lax
from jax.experimental import pallas as pl
from jax.experimental.pallas import tpu as pltpu
from jax.experimental.pallas import tpu_sc as plsc

VOCAB = 1000000
EMBED = 64
BATCH = 16384

NBLK = (VOCAB + 127) // 128
_info = plsc.get_sparse_core_info()
_NC, _NS = _info.num_cores, _info.num_subcores
NW = _NC * _NS
BLK_PER_W = NBLK // NW
EXTRA = NBLK % NW

_mesh = plsc.VectorSubcoreMesh(core_axis_name="c", subcore_axis_name="s")


@functools.partial(
    pl.kernel,
    mesh=_mesh,
    out_type=jax.ShapeDtypeStruct((BATCH * EMBED,), jnp.float32),
    scratch_types=[
        pltpu.VMEM((BATCH + 16,), jnp.int32),
        pltpu.VMEM((BATCH + 16,), jnp.int32),
        pltpu.VMEM((BATCH + 16,), jnp.int32),
        pltpu.VMEM((BATCH + 16,), jnp.int32),
        [pltpu.VMEM((EMBED, 128), jnp.float32) for _ in range(7)],
        pltpu.VMEM((32 * EMBED,), jnp.float32),
        [pltpu.SemaphoreType.DMA for _ in range(7)],
        pltpu.SemaphoreType.DMA,
    ],
    compiler_params=pltpu.CompilerParams(needs_layout_passes=False),
)
def _gather_kernel(idx_hbm, table_hbm, tail_hbm, out_hbm, idx_v, wl_v, wl_b,
                   w2_b, slabs, ring_v, sems, sem_o):
    w2_v = idx_v
    wid = lax.axis_index("s") * _NC + lax.axis_index("c")
    jstart = BLK_PER_W * wid + jnp.minimum(wid, EXTRA)
    jcount = jnp.where(wid < EXTRA, BLK_PER_W + 1, BLK_PER_W)
    jend = jstart + jcount
    iota = lax.iota(jnp.int32, 16)

    pltpu.sync_copy(idx_hbm, idx_v.at[pl.ds(0, BATCH)])

    def sel_body(g, num):
        vv = idx_v[pl.ds(g * 16, 16)]
        blk = lax.shift_right_logical(vv, 7)
        m = (blk >= jstart) & (blk < jend)
        mi = m.astype(jnp.int32)
        incl = plsc.cumsum(mi)
        pos = jnp.where(m, num + incl - mi, BATCH + 8)
        plsc.store_scatter(wl_v, [pos], vv)
        plsc.store_scatter(wl_b, [pos], g * 16 + iota)
        return num + incl[15]

    n = lax.fori_loop(0, BATCH // 16, sel_body, jnp.int32(0))
    ngrp = lax.shift_right_logical(n + 15, 4)

    bufs = [(wl_v, wl_b, w2_v, w2_b), (w2_v, w2_b, wl_v, wl_b)]
    for bit in range(8):
        sv, sb, dv, db = bufs[bit % 2]

        def mk_scan(want_one):
            def scan(g, pos):
                vv = sv[pl.ds(g * 16, 16)]
                bb = sb[pl.ds(g * 16, 16)]
                key = lax.shift_right_logical(
                    lax.shift_right_logical(vv, 7) - jstart, bit) & 1
                m = (key == want_one) & ((g * 16 + iota) < n)
                mi = m.astype(jnp.int32)
                incl = plsc.cumsum(mi)
                dst = jnp.where(m, pos + incl - mi, BATCH + 8)
                plsc.store_scatter(dv, [dst], vv)
                plsc.store_scatter(db, [dst], bb)
                return pos + incl[15]
            return scan

        z = lax.fori_loop(0, ngrp, mk_scan(0), jnp.int32(0))
        lax.fori_loop(0, ngrp, mk_scan(1), z)

    def start_fetch(j, slab, sem):
        def full(_):
            base = jnp.minimum(j, NBLK - 2) * 128
            pltpu.async_copy(table_hbm.at[:, pl.ds(base, 128)], slab, sem)
            return 0

        def tail(_):
            pltpu.async_copy(tail_hbm, slab, sem)
            return 0

        lax.cond(j >= NBLK - 1, tail, full, 0)

    def wait_fetch(j, slab, sem):
        base = jnp.minimum(j, NBLK - 2) * 128
        pltpu.make_async_copy(
            table_hbm.at[:, pl.ds(base, 128)], slab, sem).wait()

    def process_block(j, p0, slab):
        base = j * 128

        def wcond(p):
            vv = wl_v[pl.ds(p, 16)]
            return (p < n) & (lax.shift_right_logical(vv[0], 7) == j)

        def wbody(p):
            vv = wl_v[pl.ds(p, 16)]
            bb = wl_b[pl.ds(p, 16)]
            c = vv[0] - base
            b0 = bb[0]
            cvec = jnp.full((16,), 0, jnp.int32) + c
            slot = p & 31

            @pl.when(p >= 32)
            def _():
                pltpu.make_async_copy(
                    ring_v.at[pl.ds(slot * EMBED, EMBED)],
                    out_hbm.at[pl.ds(b0 * EMBED, EMBED)], sem_o).wait()

            for g2 in range(EMBED // 16):
                vals = plsc.load_gather(slab, [iota + g2 * 16, cvec])
                ring_v[pl.ds(slot * EMBED + g2 * 16, 16)] = vals
            pltpu.async_copy(
                ring_v.at[pl.ds(slot * EMBED, EMBED)],
                out_hbm.at[pl.ds(b0 * EMBED, EMBED)], sem_o)
            return p + 1

        return lax.while_loop(wcond, wbody, p0)

    NSLAB = len(slabs)
    for s in range(NSLAB):
        start_fetch(jstart + s, slabs[s], sems[s])

    def round_body(r, p):
        for s in range(NSLAB):
            j = jstart + r * NSLAB + s
            wait_fetch(j, slabs[s], sems[s])
            p = process_block(j, p, slabs[s])
            start_fetch(j + NSLAB, slabs[s], sems[s])
        return p

    nrounds = (jcount + NSLAB - 1) // NSLAB
    p_end = lax.fori_loop(0, nrounds, round_body, jnp.int32(0))
    for s in range(NSLAB):
        wait_fetch(jstart + nrounds * NSLAB + s, slabs[s], sems[s])

    def drain(r):
        pltpu.make_async_copy(
            ring_v.at[pl.ds(0, EMBED)],
            out_hbm.at[pl.ds(0, EMBED)], sem_o).wait()
        return r + 1

    lax.while_loop(lambda r: r < jnp.minimum(p_end, 32), drain, jnp.int32(0))


def kernel(centre_words, input_table):
    idx = centre_words.astype(jnp.int32)
    tail = jnp.pad(input_table[VOCAB - 64:].T, ((0, 0), (0, 64)))
    flat = _gather_kernel(idx, input_table.T, tail)
    return flat.reshape(BATCH, EMBED)
